# Initial kernel scaffold; baseline (speedup 1.0000x reference)
import functools

import jax
import jax.numpy as jnp
from jax import lax
from jax.experimental import pallas as pl
from jax.experimental.pallas import tpu as pltpu
from jax.experimental.pallas import tpu_sc as plsc

N = 10000
E = 320000
D = 128

_NC = 2
_NS = 16
_NW = _NC * _NS

_CHUNK = 80
_EDGES_PER_W = E // _NW
_CHUNKS_PER_W = _EDGES_PER_W // _CHUNK

_EPAD = 10240
_SCORE_ROWS = 2 * _EPAD
_SCORE_CHUNKS_PER_W = _SCORE_ROWS // _NW // _CHUNK

_ROWS_PER_TILE = N // _NS


def _wid():
    return lax.axis_index("s") * _NC + lax.axis_index("c")


def _sc_count_body(dst_hbm, zeros_hbm, out_hbm, didx_v, ones_v, cnt_sh):
    wid = _wid()
    sub = lax.axis_index("s")
    core = lax.axis_index("c")

    pltpu.sync_copy(zeros_hbm.at[pl.ds(sub * _ROWS_PER_TILE, _ROWS_PER_TILE)],
                    cnt_sh.at[pl.ds(sub * _ROWS_PER_TILE, _ROWS_PER_TILE)])
    for i in range(_CHUNK):
        ones_v[i, :] = jnp.full((16,), 1.0, jnp.float32)
    plsc.subcore_barrier()

    def step(i, _):
        base = pl.multiple_of(wid * _EDGES_PER_W + i * _CHUNK, _CHUNK)
        pltpu.sync_copy(dst_hbm.at[pl.ds(base, _CHUNK)], didx_v)
        pltpu.sync_copy(ones_v, cnt_sh.at[didx_v], add=True)
        return 0

    lax.fori_loop(0, _CHUNKS_PER_W, step, 0)
    plsc.subcore_barrier()
    pltpu.sync_copy(cnt_sh.at[pl.ds(sub * _ROWS_PER_TILE, _ROWS_PER_TILE)],
                    out_hbm.at[core, pl.ds(sub * _ROWS_PER_TILE, _ROWS_PER_TILE)])


def _sc_count(dst, zeros16):
    return pl.kernel(
        _sc_count_body,
        out_type=jax.ShapeDtypeStruct((_NC, N, 16), jnp.float32),
        mesh=plsc.VectorSubcoreMesh(core_axis_name="c", subcore_axis_name="s"),
        scratch_types=[
            pltpu.VMEM((_CHUNK,), jnp.int32),
            pltpu.VMEM((_CHUNK, 16), jnp.float32),
            pltpu.VMEM_SHARED((N, 16), jnp.float32),
        ],
    )(dst, zeros16)


def _sc_agg_body(h_hbm, src_hbm, dst_hbm, zeros_hbm, out_hbm,
                 sidx_v, didx_v, rows_v, acc_sh, sem):
    wid = _wid()
    sub = lax.axis_index("s")
    core = lax.axis_index("c")

    pltpu.sync_copy(zeros_hbm.at[pl.ds(sub * _ROWS_PER_TILE, _ROWS_PER_TILE)],
                    acc_sh.at[pl.ds(sub * _ROWS_PER_TILE, _ROWS_PER_TILE)])
    plsc.subcore_barrier()

    def step(i, _):
        base = pl.multiple_of(wid * _EDGES_PER_W + i * _CHUNK, _CHUNK)
        pltpu.sync_copy(src_hbm.at[pl.ds(base, _CHUNK)], sidx_v)
        pltpu.sync_copy(dst_hbm.at[pl.ds(base, _CHUNK)], didx_v)
        pltpu.async_copy(h_hbm.at[sidx_v], rows_v, sem).wait()
        pltpu.sync_copy(rows_v, acc_sh.at[didx_v], add=True)
        return 0

    lax.fori_loop(0, _CHUNKS_PER_W, step, 0)
    plsc.subcore_barrier()
    pltpu.sync_copy(acc_sh.at[pl.ds(sub * _ROWS_PER_TILE, _ROWS_PER_TILE)],
                    out_hbm.at[core, pl.ds(sub * _ROWS_PER_TILE, _ROWS_PER_TILE)])


def _sc_agg(h, src, dst, zeros128):
    return pl.kernel(
        _sc_agg_body,
        out_type=jax.ShapeDtypeStruct((_NC, N, D), jnp.float32),
        mesh=plsc.VectorSubcoreMesh(core_axis_name="c", subcore_axis_name="s"),
        scratch_types=[
            pltpu.VMEM((_CHUNK,), jnp.int32),
            pltpu.VMEM((_CHUNK,), jnp.int32),
            pltpu.VMEM((_CHUNK, D), jnp.float32),
            pltpu.VMEM_SHARED((N, D), jnp.float32),
            pltpu.SemaphoreType.DMA,
        ],
    )(h, src, dst, zeros128)


def _sc_score_gather_body(h_hbm, uidx_hbm, vidx_hbm, u_out, v_out,
                          uidx_v, vidx_v, urows_v, vrows_v, sem):
    wid = _wid()

    def step(i, _):
        base = pl.multiple_of(wid * (_SCORE_CHUNKS_PER_W * _CHUNK) + i * _CHUNK,
                              _CHUNK)
        pltpu.sync_copy(uidx_hbm.at[pl.ds(base, _CHUNK)], uidx_v)
        pltpu.sync_copy(vidx_hbm.at[pl.ds(base, _CHUNK)], vidx_v)
        pltpu.async_copy(h_hbm.at[uidx_v], urows_v, sem).wait()
        pltpu.async_copy(h_hbm.at[vidx_v], vrows_v, sem).wait()
        pltpu.sync_copy(urows_v, u_out.at[pl.ds(base, _CHUNK)])
        pltpu.sync_copy(vrows_v, v_out.at[pl.ds(base, _CHUNK)])
        return 0

    lax.fori_loop(0, _SCORE_CHUNKS_PER_W, step, 0)


def _sc_score_gather(h, uidx, vidx):
    return pl.kernel(
        _sc_score_gather_body,
        out_type=(jax.ShapeDtypeStruct((_SCORE_ROWS, D), jnp.float32),
                  jax.ShapeDtypeStruct((_SCORE_ROWS, D), jnp.float32)),
        mesh=plsc.VectorSubcoreMesh(core_axis_name="c", subcore_axis_name="s"),
        scratch_types=[
            pltpu.VMEM((_CHUNK,), jnp.int32),
            pltpu.VMEM((_CHUNK,), jnp.int32),
            pltpu.VMEM((_CHUNK, D), jnp.float32),
            pltpu.VMEM((_CHUNK, D), jnp.float32),
            pltpu.SemaphoreType.DMA,
        ],
    )(h, uidx, vidx)


_BN = 1000


def _tc_sage_body(relu, h_ref, p0_ref, p1_ref, c0_ref, c1_ref,
                  ws_ref, wn_ref, b_ref, out_ref):
    cnt = c0_ref[:, :1] + c1_ref[:, :1]
    inv = 1.0 / jnp.maximum(cnt, 1.0)
    agg = (p0_ref[...] + p1_ref[...]) * inv
    out = (jnp.dot(h_ref[...], ws_ref[...], preferred_element_type=jnp.float32)
           + jnp.dot(agg, wn_ref[...], preferred_element_type=jnp.float32)
           + b_ref[...])
    if relu:
        out = jnp.maximum(out, 0.0)
    out_ref[...] = out


def _tc_sage(h, parts, cnts, Ws, Wn, b, relu):
    grid = (N // _BN,)
    return pl.pallas_call(
        functools.partial(_tc_sage_body, relu),
        grid=grid,
        in_specs=[
            pl.BlockSpec((_BN, D), lambda i: (i, 0)),
            pl.BlockSpec((_BN, D), lambda i: (i, 0)),
            pl.BlockSpec((_BN, D), lambda i: (i, 0)),
            pl.BlockSpec((_BN, 16), lambda i: (i, 0)),
            pl.BlockSpec((_BN, 16), lambda i: (i, 0)),
            pl.BlockSpec((D, D), lambda i: (0, 0)),
            pl.BlockSpec((D, D), lambda i: (0, 0)),
            pl.BlockSpec((1, D), lambda i: (0, 0)),
        ],
        out_specs=pl.BlockSpec((_BN, D), lambda i: (i, 0)),
        out_shape=jax.ShapeDtypeStruct((N, D), jnp.float32),
    )(h, parts[0], parts[1], cnts[0], cnts[1], Ws, Wn, b)


_BS = 2048


def _tc_dot_body(u_ref, v_ref, out_ref):
    out_ref[...] = jnp.sum(u_ref[...] * v_ref[...], axis=-1, keepdims=True)


def _tc_dot(u_rows, v_rows):
    grid = (_SCORE_ROWS // _BS,)
    return pl.pallas_call(
        _tc_dot_body,
        grid=grid,
        in_specs=[
            pl.BlockSpec((_BS, D), lambda i: (i, 0)),
            pl.BlockSpec((_BS, D), lambda i: (i, 0)),
        ],
        out_specs=pl.BlockSpec((_BS, 1), lambda i: (i, 0)),
        out_shape=jax.ShapeDtypeStruct((_SCORE_ROWS, 1), jnp.float32),
    )(u_rows, v_rows)


def kernel(x, edge_index, pos_edge_index, neg_edge_index,
           W1n, W1s, b1, W2n, W2s, b2, W3n, W3s, b3):
    src = edge_index[0]
    dst = edge_index[1]
    zeros128 = jnp.zeros((N, D), jnp.float32)
    zeros16 = jnp.zeros((N, 16), jnp.float32)

    cnts = _sc_count(dst, zeros16)

    h = x
    for (Wn, Ws, b, relu) in ((W1n, W1s, b1, True),
                              (W2n, W2s, b2, True),
                              (W3n, W3s, b3, False)):
        parts = _sc_agg(h, src, dst, zeros128)
        h = _tc_sage(h, parts, cnts, Ws, Wn, b.reshape(1, D), relu)

    pad = jnp.zeros((_EPAD - pos_edge_index.shape[1],), jnp.int32)
    uidx = jnp.concatenate([pos_edge_index[0], pad, neg_edge_index[0], pad])
    vidx = jnp.concatenate([pos_edge_index[1], pad, neg_edge_index[1], pad])

    u_rows, v_rows = _sc_score_gather(h, uidx, vidx)
    scores = _tc_dot(u_rows, v_rows)

    pos_s = scores[:pos_edge_index.shape[1]]
    neg_s = scores[_EPAD:_EPAD + neg_edge_index.shape[1]]
    return (pos_s, neg_s)


# speedup vs baseline: 3.7635x; 3.7635x over previous
import functools

import jax
import jax.numpy as jnp
from jax import lax
from jax.experimental import pallas as pl
from jax.experimental.pallas import tpu as pltpu
from jax.experimental.pallas import tpu_sc as plsc

N = 10000
E = 320000
D = 128
NP = 10240

_NC = 2
_NS = 16
_NW = _NC * _NS

_CHUNK = 80
_EDGES_PER_W = E // _NW
_CHUNKS_PER_W = _EDGES_PER_W // _CHUNK

_EPAD = 10240
_SCORE_ROWS = 2 * _EPAD
_SCORE_CHUNKS_PER_W = _SCORE_ROWS // _NW // _CHUNK

_ROWS_PER_TILE = NP // _NS


def _wid():
    return lax.axis_index("s") * _NC + lax.axis_index("c")


def _sc_count_body(dst_hbm, zeros_hbm, out_hbm, didx_v, ones_v, cnt_sh):
    wid = _wid()
    sub = lax.axis_index("s")
    core = lax.axis_index("c")

    pltpu.sync_copy(zeros_hbm.at[pl.ds(sub * _ROWS_PER_TILE, _ROWS_PER_TILE)],
                    cnt_sh.at[pl.ds(sub * _ROWS_PER_TILE, _ROWS_PER_TILE)])
    for i in range(_CHUNK):
        ones_v[i, :] = jnp.full((16,), 1.0, jnp.float32)
    plsc.subcore_barrier()

    def step(i, _):
        base = pl.multiple_of(wid * _EDGES_PER_W + i * _CHUNK, _CHUNK)
        pltpu.sync_copy(dst_hbm.at[pl.ds(base, _CHUNK)], didx_v)
        pltpu.sync_copy(ones_v, cnt_sh.at[didx_v], add=True)
        return 0

    lax.fori_loop(0, _CHUNKS_PER_W, step, 0)
    plsc.subcore_barrier()
    pltpu.sync_copy(cnt_sh.at[pl.ds(sub * _ROWS_PER_TILE, _ROWS_PER_TILE)],
                    out_hbm.at[core, pl.ds(sub * _ROWS_PER_TILE, _ROWS_PER_TILE)])


def _sc_count(dst, zeros16):
    return pl.kernel(
        _sc_count_body,
        out_type=jax.ShapeDtypeStruct((_NC, NP, 16), jnp.float32),
        mesh=plsc.VectorSubcoreMesh(core_axis_name="c", subcore_axis_name="s", num_cores=_NC, num_subcores=_NS),
        scratch_types=[
            pltpu.VMEM((_CHUNK,), jnp.int32),
            pltpu.VMEM((_CHUNK, 16), jnp.float32),
            pltpu.VMEM_SHARED((NP, 16), jnp.float32),
        ],
    )(dst, zeros16)


def _sc_agg_body(h_hbm, src_hbm, dst_hbm, zeros_hbm, out_hbm,
                 sidx_v, didx_v, rows_v, acc_sh, sem):
    wid = _wid()
    sub = lax.axis_index("s")
    core = lax.axis_index("c")

    pltpu.sync_copy(zeros_hbm.at[pl.ds(sub * _ROWS_PER_TILE, _ROWS_PER_TILE)],
                    acc_sh.at[pl.ds(sub * _ROWS_PER_TILE, _ROWS_PER_TILE)])
    plsc.subcore_barrier()

    def step(i, _):
        base = pl.multiple_of(wid * _EDGES_PER_W + i * _CHUNK, _CHUNK)
        pltpu.sync_copy(src_hbm.at[pl.ds(base, _CHUNK)], sidx_v)
        pltpu.sync_copy(dst_hbm.at[pl.ds(base, _CHUNK)], didx_v)
        pltpu.async_copy(h_hbm.at[sidx_v], rows_v, sem).wait()
        pltpu.sync_copy(rows_v, acc_sh.at[didx_v], add=True)
        return 0

    lax.fori_loop(0, _CHUNKS_PER_W, step, 0)
    plsc.subcore_barrier()
    pltpu.sync_copy(acc_sh.at[pl.ds(sub * _ROWS_PER_TILE, _ROWS_PER_TILE)],
                    out_hbm.at[core, pl.ds(sub * _ROWS_PER_TILE, _ROWS_PER_TILE)])


def _sc_agg(h, src, dst, zeros128):
    return pl.kernel(
        _sc_agg_body,
        out_type=jax.ShapeDtypeStruct((_NC, NP, D), jnp.float32),
        mesh=plsc.VectorSubcoreMesh(core_axis_name="c", subcore_axis_name="s", num_cores=_NC, num_subcores=_NS),
        scratch_types=[
            pltpu.VMEM((_CHUNK,), jnp.int32),
            pltpu.VMEM((_CHUNK,), jnp.int32),
            pltpu.VMEM((_CHUNK, D), jnp.float32),
            pltpu.VMEM_SHARED((NP, D), jnp.float32),
            pltpu.SemaphoreType.DMA,
        ],
    )(h, src, dst, zeros128)


def _sc_score_gather_body(h_hbm, uidx_hbm, vidx_hbm, u_out, v_out,
                          uidx_v, vidx_v, urows_v, vrows_v, sem):
    wid = _wid()

    def step(i, _):
        base = pl.multiple_of(wid * (_SCORE_CHUNKS_PER_W * _CHUNK) + i * _CHUNK,
                              _CHUNK)
        pltpu.sync_copy(uidx_hbm.at[pl.ds(base, _CHUNK)], uidx_v)
        pltpu.sync_copy(vidx_hbm.at[pl.ds(base, _CHUNK)], vidx_v)
        pltpu.async_copy(h_hbm.at[uidx_v], urows_v, sem).wait()
        pltpu.async_copy(h_hbm.at[vidx_v], vrows_v, sem).wait()
        pltpu.sync_copy(urows_v, u_out.at[pl.ds(base, _CHUNK)])
        pltpu.sync_copy(vrows_v, v_out.at[pl.ds(base, _CHUNK)])
        return 0

    lax.fori_loop(0, _SCORE_CHUNKS_PER_W, step, 0)


def _sc_score_gather(h, uidx, vidx):
    return pl.kernel(
        _sc_score_gather_body,
        out_type=(jax.ShapeDtypeStruct((_SCORE_ROWS, D), jnp.float32),
                  jax.ShapeDtypeStruct((_SCORE_ROWS, D), jnp.float32)),
        mesh=plsc.VectorSubcoreMesh(core_axis_name="c", subcore_axis_name="s", num_cores=_NC, num_subcores=_NS),
        scratch_types=[
            pltpu.VMEM((_CHUNK,), jnp.int32),
            pltpu.VMEM((_CHUNK,), jnp.int32),
            pltpu.VMEM((_CHUNK, D), jnp.float32),
            pltpu.VMEM((_CHUNK, D), jnp.float32),
            pltpu.SemaphoreType.DMA,
        ],
    )(h, uidx, vidx)


_BN = 1024


def _tc_sage_body(relu, h_ref, p0_ref, p1_ref, c0_ref, c1_ref,
                  ws_ref, wn_ref, b_ref, out_ref):
    cnt = c0_ref[:, :1] + c1_ref[:, :1]
    inv = 1.0 / jnp.maximum(cnt, 1.0)
    agg = (p0_ref[...] + p1_ref[...]) * inv
    out = (jnp.dot(h_ref[...], ws_ref[...], preferred_element_type=jnp.float32)
           + jnp.dot(agg, wn_ref[...], preferred_element_type=jnp.float32)
           + b_ref[...])
    if relu:
        out = jnp.maximum(out, 0.0)
    out_ref[...] = out


def _tc_sage(h, parts, cnts, Ws, Wn, b, relu):
    grid = (NP // _BN,)
    return pl.pallas_call(
        functools.partial(_tc_sage_body, relu),
        grid=grid,
        in_specs=[
            pl.BlockSpec((_BN, D), lambda i: (i, 0)),
            pl.BlockSpec((_BN, D), lambda i: (i, 0)),
            pl.BlockSpec((_BN, D), lambda i: (i, 0)),
            pl.BlockSpec((_BN, 16), lambda i: (i, 0)),
            pl.BlockSpec((_BN, 16), lambda i: (i, 0)),
            pl.BlockSpec((D, D), lambda i: (0, 0)),
            pl.BlockSpec((D, D), lambda i: (0, 0)),
            pl.BlockSpec((1, D), lambda i: (0, 0)),
        ],
        out_specs=pl.BlockSpec((_BN, D), lambda i: (i, 0)),
        out_shape=jax.ShapeDtypeStruct((NP, D), jnp.float32),
    )(h, parts[0], parts[1], cnts[0], cnts[1], Ws, Wn, b)


_BS = 2048


def _tc_dot_body(u_ref, v_ref, out_ref):
    out_ref[...] = jnp.sum(u_ref[...] * v_ref[...], axis=-1, keepdims=True)


def _tc_dot(u_rows, v_rows):
    grid = (_SCORE_ROWS // _BS,)
    return pl.pallas_call(
        _tc_dot_body,
        grid=grid,
        in_specs=[
            pl.BlockSpec((_BS, D), lambda i: (i, 0)),
            pl.BlockSpec((_BS, D), lambda i: (i, 0)),
        ],
        out_specs=pl.BlockSpec((_BS, 1), lambda i: (i, 0)),
        out_shape=jax.ShapeDtypeStruct((_SCORE_ROWS, 1), jnp.float32),
    )(u_rows, v_rows)


def kernel(x, edge_index, pos_edge_index, neg_edge_index,
           W1n, W1s, b1, W2n, W2s, b2, W3n, W3s, b3):
    src = edge_index[0]
    dst = edge_index[1]
    zeros128 = jnp.zeros((NP, D), jnp.float32)
    zeros16 = jnp.zeros((NP, 16), jnp.float32)

    ones_nodes = jnp.ones((NP, D), jnp.float32)
    cnt_parts = _sc_agg(ones_nodes, src, dst, zeros128)
    cnts = (cnt_parts[0][:, :16], cnt_parts[1][:, :16])

    h = jnp.pad(x, ((0, NP - N), (0, 0)))
    for (Wn, Ws, b, relu) in ((W1n, W1s, b1, True),
                              (W2n, W2s, b2, True),
                              (W3n, W3s, b3, False)):
        parts = _sc_agg(h, src, dst, zeros128)
        h = _tc_sage(h, parts, cnts, Ws, Wn, b.reshape(1, D), relu)

    pad = jnp.zeros((_EPAD - pos_edge_index.shape[1],), jnp.int32)
    uidx = jnp.concatenate([pos_edge_index[0], pad, neg_edge_index[0], pad])
    vidx = jnp.concatenate([pos_edge_index[1], pad, neg_edge_index[1], pad])

    u_rows, v_rows = _sc_score_gather(h, uidx, vidx)
    scores = _tc_dot(u_rows, v_rows)

    pos_s = scores[:pos_edge_index.shape[1]]
    neg_s = scores[_EPAD:_EPAD + neg_edge_index.shape[1]]
    return (pos_s, neg_s)


# speedup vs baseline: 4.5408x; 1.2065x over previous
import functools

import jax
import jax.numpy as jnp
from jax import lax
from jax.experimental import pallas as pl
from jax.experimental.pallas import tpu as pltpu
from jax.experimental.pallas import tpu_sc as plsc

N = 10000
E = 320000
D = 128
NP = 10240

_NC = 2
_NS = 16
_NW = _NC * _NS

_CHUNK = 40
_CCHUNK = 80
_SCHUNK = 80
_EDGES_PER_W = E // _NW
_CHUNKS_PER_W = _EDGES_PER_W // _CHUNK
_CCHUNKS_PER_W = _EDGES_PER_W // _CCHUNK

_EPAD = 10240
_SCORE_ROWS = 2 * _EPAD
_SCORE_CHUNKS_PER_W = _SCORE_ROWS // _NW // _SCHUNK

_ROWS_PER_TILE = NP // _NS


def _wid():
    return lax.axis_index("s") * _NC + lax.axis_index("c")


def _sc_count_body(dst_hbm, ones_hbm, zeros_hbm, out_hbm, didx_v, ones_v, cnt_sh):
    wid = _wid()
    sub = lax.axis_index("s")
    core = lax.axis_index("c")

    pltpu.sync_copy(zeros_hbm.at[pl.ds(sub * _ROWS_PER_TILE, _ROWS_PER_TILE)],
                    cnt_sh.at[pl.ds(sub * _ROWS_PER_TILE, _ROWS_PER_TILE)])
    pltpu.sync_copy(ones_hbm, ones_v)
    plsc.subcore_barrier()

    def step(i, _):
        base = pl.multiple_of(wid * _EDGES_PER_W + i * _CCHUNK, _CCHUNK)
        pltpu.sync_copy(dst_hbm.at[pl.ds(base, _CCHUNK)], didx_v)
        pltpu.sync_copy(ones_v, cnt_sh.at[didx_v], add=True)
        return 0

    lax.fori_loop(0, _CCHUNKS_PER_W, step, 0)
    plsc.subcore_barrier()
    pltpu.sync_copy(cnt_sh.at[pl.ds(sub * _ROWS_PER_TILE, _ROWS_PER_TILE)],
                    out_hbm.at[core, pl.ds(sub * _ROWS_PER_TILE, _ROWS_PER_TILE)])


def _sc_count(dst, ones_chunk, zeros128):
    return pl.kernel(
        _sc_count_body,
        out_type=jax.ShapeDtypeStruct((_NC, NP, D), jnp.float32),
        mesh=plsc.VectorSubcoreMesh(core_axis_name="c", subcore_axis_name="s", num_cores=_NC, num_subcores=_NS),
        scratch_types=[
            pltpu.VMEM((_CCHUNK,), jnp.int32),
            pltpu.VMEM((_CCHUNK, D), jnp.float32),
            pltpu.VMEM_SHARED((NP, D), jnp.float32),
        ],
    )(dst, ones_chunk, zeros128)


_NBUF = 5
assert _CHUNKS_PER_W % _NBUF == 0
_STEADY_CHUNKS = _CHUNKS_PER_W - _NBUF


def _sc_agg_body(h_hbm, src_hbm, dst_hbm, zeros_hbm, out_hbm, *refs):
    sidx_b = refs[0:_NBUF]
    didx_b = refs[_NBUF:2 * _NBUF]
    rows_b = refs[2 * _NBUF:3 * _NBUF]
    acc_sh = refs[3 * _NBUF]
    sems = refs[3 * _NBUF + 1:]

    wid = _wid()
    sub = lax.axis_index("s")
    core = lax.axis_index("c")
    ebase = wid * _EDGES_PER_W

    pltpu.sync_copy(zeros_hbm.at[pl.ds(sub * _ROWS_PER_TILE, _ROWS_PER_TILE)],
                    acc_sh.at[pl.ds(sub * _ROWS_PER_TILE, _ROWS_PER_TILE)])
    plsc.subcore_barrier()

    for b in range(_NBUF):
        base = pl.multiple_of(ebase + b * _CHUNK, _CHUNK)
        pltpu.sync_copy(src_hbm.at[pl.ds(base, _CHUNK)], sidx_b[b])
        pltpu.sync_copy(dst_hbm.at[pl.ds(base, _CHUNK)], didx_b[b])
        pltpu.async_copy(h_hbm.at[sidx_b[b]], rows_b[b], sems[b])

    def step(g, _):
        for b in range(_NBUF):
            i = g + b
            pltpu.make_async_copy(h_hbm.at[sidx_b[b]], rows_b[b], sems[b]).wait()
            pltpu.sync_copy(rows_b[b], acc_sh.at[didx_b[b]], add=True)
            base = pl.multiple_of(ebase + (i + _NBUF) * _CHUNK, _CHUNK)
            pltpu.sync_copy(src_hbm.at[pl.ds(base, _CHUNK)], sidx_b[b])
            pltpu.sync_copy(dst_hbm.at[pl.ds(base, _CHUNK)], didx_b[b])
            pltpu.async_copy(h_hbm.at[sidx_b[b]], rows_b[b], sems[b])
        return 0

    lax.fori_loop(0, _STEADY_CHUNKS // _NBUF, lambda g, c: step(g * _NBUF, c), 0)

    for b in range(_NBUF):
        pltpu.make_async_copy(h_hbm.at[sidx_b[b]], rows_b[b], sems[b]).wait()
        pltpu.sync_copy(rows_b[b], acc_sh.at[didx_b[b]], add=True)

    plsc.subcore_barrier()
    pltpu.sync_copy(acc_sh.at[pl.ds(sub * _ROWS_PER_TILE, _ROWS_PER_TILE)],
                    out_hbm.at[core, pl.ds(sub * _ROWS_PER_TILE, _ROWS_PER_TILE)])


def _sc_agg(h, src, dst, zeros128):
    return pl.kernel(
        _sc_agg_body,
        out_type=jax.ShapeDtypeStruct((_NC, NP, D), jnp.float32),
        mesh=plsc.VectorSubcoreMesh(core_axis_name="c", subcore_axis_name="s", num_cores=_NC, num_subcores=_NS),
        scratch_types=(
            [pltpu.VMEM((_CHUNK,), jnp.int32) for _ in range(2 * _NBUF)]
            + [pltpu.VMEM((_CHUNK, D), jnp.float32) for _ in range(_NBUF)]
            + [pltpu.VMEM_SHARED((NP, D), jnp.float32)]
            + [pltpu.SemaphoreType.DMA for _ in range(_NBUF)]
        ),
    )(h, src, dst, zeros128)


def _sc_score_gather_body(h_hbm, uidx_hbm, vidx_hbm, u_out, v_out,
                          uidx_v, vidx_v, urows_v, vrows_v, sem):
    wid = _wid()

    def step(i, _):
        base = pl.multiple_of(wid * (_SCORE_CHUNKS_PER_W * _SCHUNK) + i * _SCHUNK,
                              _SCHUNK)
        pltpu.sync_copy(uidx_hbm.at[pl.ds(base, _SCHUNK)], uidx_v)
        pltpu.sync_copy(vidx_hbm.at[pl.ds(base, _SCHUNK)], vidx_v)
        pltpu.async_copy(h_hbm.at[uidx_v], urows_v, sem).wait()
        pltpu.async_copy(h_hbm.at[vidx_v], vrows_v, sem).wait()
        pltpu.sync_copy(urows_v, u_out.at[pl.ds(base, _SCHUNK)])
        pltpu.sync_copy(vrows_v, v_out.at[pl.ds(base, _SCHUNK)])
        return 0

    lax.fori_loop(0, _SCORE_CHUNKS_PER_W, step, 0)


def _sc_score_gather(h, uidx, vidx):
    return pl.kernel(
        _sc_score_gather_body,
        out_type=(jax.ShapeDtypeStruct((_SCORE_ROWS, D), jnp.float32),
                  jax.ShapeDtypeStruct((_SCORE_ROWS, D), jnp.float32)),
        mesh=plsc.VectorSubcoreMesh(core_axis_name="c", subcore_axis_name="s", num_cores=_NC, num_subcores=_NS),
        scratch_types=[
            pltpu.VMEM((_SCHUNK,), jnp.int32),
            pltpu.VMEM((_SCHUNK,), jnp.int32),
            pltpu.VMEM((_SCHUNK, D), jnp.float32),
            pltpu.VMEM((_SCHUNK, D), jnp.float32),
            pltpu.SemaphoreType.DMA,
        ],
    )(h, uidx, vidx)


_BN = 1024


def _tc_sage_body(relu, h_ref, p0_ref, p1_ref, c0_ref, c1_ref,
                  ws_ref, wn_ref, b_ref, out_ref):
    cnt = c0_ref[:, :1] + c1_ref[:, :1]
    inv = 1.0 / jnp.maximum(cnt, 1.0)
    agg = (p0_ref[...] + p1_ref[...]) * inv
    out = (jnp.dot(h_ref[...], ws_ref[...], preferred_element_type=jnp.float32)
           + jnp.dot(agg, wn_ref[...], preferred_element_type=jnp.float32)
           + b_ref[...])
    if relu:
        out = jnp.maximum(out, 0.0)
    out_ref[...] = out


def _tc_sage(h, parts, cnts, Ws, Wn, b, relu):
    grid = (NP // _BN,)
    return pl.pallas_call(
        functools.partial(_tc_sage_body, relu),
        grid=grid,
        in_specs=[
            pl.BlockSpec((_BN, D), lambda i: (i, 0)),
            pl.BlockSpec((_BN, D), lambda i: (i, 0)),
            pl.BlockSpec((_BN, D), lambda i: (i, 0)),
            pl.BlockSpec((_BN, 16), lambda i: (i, 0)),
            pl.BlockSpec((_BN, 16), lambda i: (i, 0)),
            pl.BlockSpec((D, D), lambda i: (0, 0)),
            pl.BlockSpec((D, D), lambda i: (0, 0)),
            pl.BlockSpec((1, D), lambda i: (0, 0)),
        ],
        out_specs=pl.BlockSpec((_BN, D), lambda i: (i, 0)),
        out_shape=jax.ShapeDtypeStruct((NP, D), jnp.float32),
    )(h, parts[0], parts[1], cnts[0], cnts[1], Ws, Wn, b)


_BS = 2048


def _tc_dot_body(u_ref, v_ref, out_ref):
    out_ref[...] = jnp.sum(u_ref[...] * v_ref[...], axis=-1, keepdims=True)


def _tc_dot(u_rows, v_rows):
    grid = (_SCORE_ROWS // _BS,)
    return pl.pallas_call(
        _tc_dot_body,
        grid=grid,
        in_specs=[
            pl.BlockSpec((_BS, D), lambda i: (i, 0)),
            pl.BlockSpec((_BS, D), lambda i: (i, 0)),
        ],
        out_specs=pl.BlockSpec((_BS, 1), lambda i: (i, 0)),
        out_shape=jax.ShapeDtypeStruct((_SCORE_ROWS, 1), jnp.float32),
    )(u_rows, v_rows)


def kernel(x, edge_index, pos_edge_index, neg_edge_index,
           W1n, W1s, b1, W2n, W2s, b2, W3n, W3s, b3):
    src = edge_index[0]
    dst = edge_index[1]
    zeros128 = jnp.zeros((NP, D), jnp.float32)
    ones_chunk = jnp.ones((_CCHUNK, D), jnp.float32)

    cnt_parts = _sc_count(dst, ones_chunk, zeros128)
    cnts = (cnt_parts[0][:, :16], cnt_parts[1][:, :16])

    h = jnp.pad(x, ((0, NP - N), (0, 0)))
    for (Wn, Ws, b, relu) in ((W1n, W1s, b1, True),
                              (W2n, W2s, b2, True),
                              (W3n, W3s, b3, False)):
        parts = _sc_agg(h, src, dst, zeros128)
        h = _tc_sage(h, parts, cnts, Ws, Wn, b.reshape(1, D), relu)

    pad = jnp.zeros((_EPAD - pos_edge_index.shape[1],), jnp.int32)
    uidx = jnp.concatenate([pos_edge_index[0], pad, neg_edge_index[0], pad])
    vidx = jnp.concatenate([pos_edge_index[1], pad, neg_edge_index[1], pad])

    u_rows, v_rows = _sc_score_gather(h, uidx, vidx)
    scores = _tc_dot(u_rows, v_rows)

    pos_s = scores[:pos_edge_index.shape[1]]
    neg_s = scores[_EPAD:_EPAD + neg_edge_index.shape[1]]
    return (pos_s, neg_s)


# speedup vs baseline: 6.1936x; 1.3640x over previous
import functools

import jax
import jax.numpy as jnp
from jax import lax
from jax.experimental import pallas as pl
from jax.experimental.pallas import tpu as pltpu
from jax.experimental.pallas import tpu_sc as plsc

N = 10000
E = 320000
D = 128
NP = 10240

_NC = 2
_NS = 16
_NW = _NC * _NS

_CHUNK = 80
_EDGES_PER_W = E // _NW
_CHUNKS_PER_W = _EDGES_PER_W // _CHUNK

_EPAD = 10240
_SCORE_ROWS = 2 * _EPAD
_SCORE_CHUNKS_PER_W = _SCORE_ROWS // _NW // _CHUNK

_ROWS_PER_TILE = NP // _NS


def _wid():
    return lax.axis_index("s") * _NC + lax.axis_index("c")


def _sc_count_body(dst_hbm, ones_hbm, zeros_hbm, out_hbm, didx_v, ones_v, cnt_sh):
    wid = _wid()
    sub = lax.axis_index("s")
    core = lax.axis_index("c")

    pltpu.sync_copy(zeros_hbm.at[pl.ds(sub * _ROWS_PER_TILE, _ROWS_PER_TILE)],
                    cnt_sh.at[pl.ds(sub * _ROWS_PER_TILE, _ROWS_PER_TILE)])
    pltpu.sync_copy(ones_hbm, ones_v)
    plsc.subcore_barrier()

    def step(i, _):
        base = pl.multiple_of(wid * _EDGES_PER_W + i * _CHUNK, _CHUNK)
        pltpu.sync_copy(dst_hbm.at[pl.ds(base, _CHUNK)], didx_v)
        pltpu.sync_copy(ones_v, cnt_sh.at[didx_v], add=True)
        return 0

    lax.fori_loop(0, _CHUNKS_PER_W, step, 0)
    plsc.subcore_barrier()
    pltpu.sync_copy(cnt_sh.at[pl.ds(sub * _ROWS_PER_TILE, _ROWS_PER_TILE)],
                    out_hbm.at[core, pl.ds(sub * _ROWS_PER_TILE, _ROWS_PER_TILE)])


def _sc_count(dst, ones_chunk, zeros_f):
    return pl.kernel(
        _sc_count_body,
        out_type=jax.ShapeDtypeStruct((_NC, NP, D), jnp.float32),
        mesh=plsc.VectorSubcoreMesh(core_axis_name="c", subcore_axis_name="s", num_cores=_NC, num_subcores=_NS),
        scratch_types=[
            pltpu.VMEM((_CHUNK,), jnp.int32),
            pltpu.VMEM((_CHUNK, D), jnp.float32),
            pltpu.VMEM_SHARED((NP, D), jnp.float32),
        ],
    )(dst, ones_chunk, zeros_f)


_NBUF = 4
_STEADY_GROUPS = (_CHUNKS_PER_W - 1) // _NBUF - 1
_DRAIN_BASE = _NBUF * (_STEADY_GROUPS + 1)


def _sc_agg_body(h_hbm, src_hbm, dst_hbm, zeros_hbm, out_hbm, *refs):
    sidx_b = refs[0:_NBUF]
    didx_b = refs[_NBUF:2 * _NBUF]
    rows_b = refs[2 * _NBUF:3 * _NBUF]
    acc_sh = refs[3 * _NBUF]
    sems = refs[3 * _NBUF + 1:]

    wid = _wid()
    sub = lax.axis_index("s")
    core = lax.axis_index("c")
    ebase = wid * _EDGES_PER_W

    pltpu.sync_copy(zeros_hbm.at[pl.ds(sub * _ROWS_PER_TILE, _ROWS_PER_TILE)],
                    acc_sh.at[pl.ds(sub * _ROWS_PER_TILE, _ROWS_PER_TILE)])
    plsc.subcore_barrier()

    for b in range(_NBUF):
        base = pl.multiple_of(ebase + b * _CHUNK, _CHUNK)
        pltpu.sync_copy(src_hbm.at[pl.ds(base, _CHUNK)], sidx_b[b])
        pltpu.sync_copy(dst_hbm.at[pl.ds(base, _CHUNK)], didx_b[b])
        pltpu.async_copy(h_hbm.at[sidx_b[b]], rows_b[b], sems[b])

    def step(g, _):
        for b in range(_NBUF):
            i = g + b
            pltpu.make_async_copy(h_hbm.at[sidx_b[b]], rows_b[b], sems[b]).wait()
            pltpu.sync_copy(rows_b[b], acc_sh.at[didx_b[b]], add=True)
            base = pl.multiple_of(ebase + (i + _NBUF) * _CHUNK, _CHUNK)
            pltpu.sync_copy(src_hbm.at[pl.ds(base, _CHUNK)], sidx_b[b])
            pltpu.sync_copy(dst_hbm.at[pl.ds(base, _CHUNK)], didx_b[b])
            pltpu.async_copy(h_hbm.at[sidx_b[b]], rows_b[b], sems[b])
        return 0

    lax.fori_loop(0, _STEADY_GROUPS, lambda g, c: step(g * _NBUF, c), 0)

    for b in range(_NBUF):
        pltpu.make_async_copy(h_hbm.at[sidx_b[b]], rows_b[b], sems[b]).wait()
        pltpu.sync_copy(rows_b[b], acc_sh.at[didx_b[b]], add=True)

    base = pl.multiple_of(ebase + _DRAIN_BASE * _CHUNK, _CHUNK)
    pltpu.sync_copy(src_hbm.at[pl.ds(base, _CHUNK)], sidx_b[0])
    pltpu.sync_copy(dst_hbm.at[pl.ds(base, _CHUNK)], didx_b[0])
    pltpu.async_copy(h_hbm.at[sidx_b[0]], rows_b[0], sems[0])
    pltpu.make_async_copy(h_hbm.at[sidx_b[0]], rows_b[0], sems[0]).wait()
    pltpu.sync_copy(rows_b[0], acc_sh.at[didx_b[0]], add=True)

    plsc.subcore_barrier()
    pltpu.sync_copy(acc_sh.at[pl.ds(sub * _ROWS_PER_TILE, _ROWS_PER_TILE)],
                    out_hbm.at[core, pl.ds(sub * _ROWS_PER_TILE, _ROWS_PER_TILE)])


def _sc_agg(h, src, dst, zeros_f):
    return pl.kernel(
        _sc_agg_body,
        out_type=jax.ShapeDtypeStruct((_NC, NP, D), jnp.float32),
        mesh=plsc.VectorSubcoreMesh(core_axis_name="c", subcore_axis_name="s", num_cores=_NC, num_subcores=_NS),
        scratch_types=(
            [pltpu.VMEM((_CHUNK,), jnp.int32) for _ in range(2 * _NBUF)]
            + [pltpu.VMEM((_CHUNK, D), jnp.float32) for _ in range(_NBUF)]
            + [pltpu.VMEM_SHARED((NP, D), jnp.float32)]
            + [pltpu.SemaphoreType.DMA for _ in range(_NBUF)]
        ),
    )(h, src, dst, zeros_f)


def _sc_score_gather_body(h_hbm, uidx_hbm, vidx_hbm, u_out, v_out,
                          uidx_v, vidx_v, urows_v, vrows_v, sem):
    wid = _wid()

    def step(i, _):
        base = pl.multiple_of(wid * (_SCORE_CHUNKS_PER_W * _CHUNK) + i * _CHUNK,
                              _CHUNK)
        pltpu.sync_copy(uidx_hbm.at[pl.ds(base, _CHUNK)], uidx_v)
        pltpu.sync_copy(vidx_hbm.at[pl.ds(base, _CHUNK)], vidx_v)
        pltpu.async_copy(h_hbm.at[uidx_v], urows_v, sem).wait()
        pltpu.async_copy(h_hbm.at[vidx_v], vrows_v, sem).wait()
        pltpu.sync_copy(urows_v, u_out.at[pl.ds(base, _CHUNK)])
        pltpu.sync_copy(vrows_v, v_out.at[pl.ds(base, _CHUNK)])
        return 0

    lax.fori_loop(0, _SCORE_CHUNKS_PER_W, step, 0)


def _sc_score_gather(h, uidx, vidx):
    return pl.kernel(
        _sc_score_gather_body,
        out_type=(jax.ShapeDtypeStruct((_SCORE_ROWS, D), jnp.float32),
                  jax.ShapeDtypeStruct((_SCORE_ROWS, D), jnp.float32)),
        mesh=plsc.VectorSubcoreMesh(core_axis_name="c", subcore_axis_name="s", num_cores=_NC, num_subcores=_NS),
        scratch_types=[
            pltpu.VMEM((_CHUNK,), jnp.int32),
            pltpu.VMEM((_CHUNK,), jnp.int32),
            pltpu.VMEM((_CHUNK, D), jnp.float32),
            pltpu.VMEM((_CHUNK, D), jnp.float32),
            pltpu.SemaphoreType.DMA,
        ],
    )(h, uidx, vidx)


_BN = 1024


def _tc_sage_body(relu, h_ref, p0_ref, p1_ref, c0_ref, c1_ref,
                  ws_ref, wn_ref, b_ref, out_ref):
    cnt = c0_ref[:, :1] + c1_ref[:, :1]
    inv = 1.0 / jnp.maximum(cnt, 1.0)
    agg = (p0_ref[...] + p1_ref[...]) * inv
    out = (jnp.dot(h_ref[...], ws_ref[...], preferred_element_type=jnp.float32)
           + jnp.dot(agg, wn_ref[...], preferred_element_type=jnp.float32)
           + b_ref[...])
    if relu:
        out = jnp.maximum(out, 0.0)
    out_ref[...] = out


def _tc_sage(h, parts, cnts, Ws, Wn, b, relu):
    grid = (NP // _BN,)
    return pl.pallas_call(
        functools.partial(_tc_sage_body, relu),
        grid=grid,
        in_specs=[
            pl.BlockSpec((_BN, D), lambda i: (i, 0)),
            pl.BlockSpec((_BN, D), lambda i: (i, 0)),
            pl.BlockSpec((_BN, D), lambda i: (i, 0)),
            pl.BlockSpec((_BN, 16), lambda i: (i, 0)),
            pl.BlockSpec((_BN, 16), lambda i: (i, 0)),
            pl.BlockSpec((D, D), lambda i: (0, 0)),
            pl.BlockSpec((D, D), lambda i: (0, 0)),
            pl.BlockSpec((1, D), lambda i: (0, 0)),
        ],
        out_specs=pl.BlockSpec((_BN, D), lambda i: (i, 0)),
        out_shape=jax.ShapeDtypeStruct((NP, D), jnp.float32),
    )(h, parts[0], parts[1], cnts[0], cnts[1], Ws, Wn, b)


_BS = 2048


def _tc_dot_body(u_ref, v_ref, out_ref):
    out_ref[...] = jnp.sum(u_ref[...] * v_ref[...], axis=-1, keepdims=True)


def _tc_dot(u_rows, v_rows):
    grid = (_SCORE_ROWS // _BS,)
    return pl.pallas_call(
        _tc_dot_body,
        grid=grid,
        in_specs=[
            pl.BlockSpec((_BS, D), lambda i: (i, 0)),
            pl.BlockSpec((_BS, D), lambda i: (i, 0)),
        ],
        out_specs=pl.BlockSpec((_BS, 1), lambda i: (i, 0)),
        out_shape=jax.ShapeDtypeStruct((_SCORE_ROWS, 1), jnp.float32),
    )(u_rows, v_rows)


def kernel(x, edge_index, pos_edge_index, neg_edge_index,
           W1n, W1s, b1, W2n, W2s, b2, W3n, W3s, b3):
    src = edge_index[0]
    dst = edge_index[1]
    zeros_f = jnp.zeros((NP, D), jnp.float32)
    ones_chunk = jnp.ones((_CHUNK, D), jnp.float32)

    cnt_parts = _sc_count(dst, ones_chunk, zeros_f)
    cnts = (cnt_parts[0][:, :16], cnt_parts[1][:, :16])

    h = jnp.pad(x, ((0, NP - N), (0, 0)))
    for (Wn, Ws, b, relu) in ((W1n, W1s, b1, True),
                              (W2n, W2s, b2, True),
                              (W3n, W3s, b3, False)):
        parts = _sc_agg(h, src, dst, zeros_f)
        h = _tc_sage(h, parts, cnts, Ws, Wn, b.reshape(1, D), relu)

    pad = jnp.zeros((_EPAD - pos_edge_index.shape[1],), jnp.int32)
    uidx = jnp.concatenate([pos_edge_index[0], pad, neg_edge_index[0], pad])
    vidx = jnp.concatenate([pos_edge_index[1], pad, neg_edge_index[1], pad])

    u_rows, v_rows = _sc_score_gather(h, uidx, vidx)
    scores = _tc_dot(u_rows, v_rows)

    pos_s = scores[:pos_edge_index.shape[1]]
    neg_s = scores[_EPAD:_EPAD + neg_edge_index.shape[1]]
    return (pos_s, neg_s)


# speedup vs baseline: 7.9203x; 1.2788x over previous
import functools

import jax
import jax.numpy as jnp
from jax import lax
from jax.experimental import pallas as pl
from jax.experimental.pallas import tpu as pltpu
from jax.experimental.pallas import tpu_sc as plsc

N = 10000
E = 320000
D = 128
NP = 10240

_NC = 2
_NS = 16
_NW = _NC * _NS

_CHUNK = 80
_EDGES_PER_W = E // _NW
_CHUNKS_PER_W = _EDGES_PER_W // _CHUNK

_EPAD = 10240
_SCORE_ROWS = 2 * _EPAD
_SCORE_CHUNKS_PER_W = _SCORE_ROWS // _NW // _CHUNK

_ROWS_PER_TILE = NP // _NS


def _wid():
    return lax.axis_index("s") * _NC + lax.axis_index("c")


def _sc_count_body(dst_hbm, ones_hbm, zeros_hbm, out_hbm, didx_v, ones_v, cnt_sh):
    wid = _wid()
    sub = lax.axis_index("s")
    core = lax.axis_index("c")

    pltpu.sync_copy(zeros_hbm.at[pl.ds(sub * _ROWS_PER_TILE, _ROWS_PER_TILE)],
                    cnt_sh.at[pl.ds(sub * _ROWS_PER_TILE, _ROWS_PER_TILE)])
    pltpu.sync_copy(ones_hbm, ones_v)
    plsc.subcore_barrier()

    def step(i, _):
        base = pl.multiple_of(wid * _EDGES_PER_W + i * _CHUNK, _CHUNK)
        pltpu.sync_copy(dst_hbm.at[pl.ds(base, _CHUNK)], didx_v)
        pltpu.sync_copy(ones_v, cnt_sh.at[didx_v], add=True)
        return 0

    lax.fori_loop(0, _CHUNKS_PER_W, step, 0)
    plsc.subcore_barrier()
    pltpu.sync_copy(cnt_sh.at[pl.ds(sub * _ROWS_PER_TILE, _ROWS_PER_TILE)],
                    out_hbm.at[core, pl.ds(sub * _ROWS_PER_TILE, _ROWS_PER_TILE)])


def _sc_count(dst, ones_chunk, zeros_f):
    return pl.kernel(
        _sc_count_body,
        out_type=jax.ShapeDtypeStruct((_NC, NP, D), jnp.float32),
        mesh=plsc.VectorSubcoreMesh(core_axis_name="c", subcore_axis_name="s", num_cores=_NC, num_subcores=_NS),
        scratch_types=[
            pltpu.VMEM((_CHUNK,), jnp.int32),
            pltpu.VMEM((_CHUNK, D), jnp.float32),
            pltpu.VMEM_SHARED((NP, D), jnp.float32),
        ],
    )(dst, ones_chunk, zeros_f)


_NBUF = 2
_STEADY_GROUPS = (_CHUNKS_PER_W - 1) // _NBUF - 1
_DRAIN_BASE = _NBUF * (_STEADY_GROUPS + 1)


def _sc_agg_body(h_hbm, src_hbm, dst_hbm, zeros_hbm, out_hbm, *refs):
    sidx_all = refs[0]
    didx_all = refs[1]
    rows_b = refs[2:2 + _NBUF]
    acc_sh = refs[2 + _NBUF]
    gsems = refs[3 + _NBUF:3 + 2 * _NBUF]
    ssems = refs[3 + 2 * _NBUF:3 + 3 * _NBUF]

    wid = _wid()
    sub = lax.axis_index("s")
    core = lax.axis_index("c")
    ebase = pl.multiple_of(wid * _EDGES_PER_W, _EDGES_PER_W)

    pltpu.sync_copy(zeros_hbm.at[pl.ds(sub * _ROWS_PER_TILE, _ROWS_PER_TILE)],
                    acc_sh.at[pl.ds(sub * _ROWS_PER_TILE, _ROWS_PER_TILE)])
    pltpu.sync_copy(src_hbm.at[pl.ds(ebase, _EDGES_PER_W)], sidx_all)
    pltpu.sync_copy(dst_hbm.at[pl.ds(ebase, _EDGES_PER_W)], didx_all)
    plsc.subcore_barrier()

    def sidx(i):
        return sidx_all.at[pl.ds(pl.multiple_of(i * _CHUNK, _CHUNK), _CHUNK)]

    def didx(i):
        return didx_all.at[pl.ds(pl.multiple_of(i * _CHUNK, _CHUNK), _CHUNK)]

    for b in range(_NBUF):
        pltpu.async_copy(h_hbm.at[sidx(b)], rows_b[b], gsems[b])

    def step(g, _):
        for b in range(_NBUF):
            i = g + b
            pltpu.make_async_copy(h_hbm.at[sidx(i)], rows_b[b], gsems[b]).wait()
            pltpu.async_copy(rows_b[b], acc_sh.at[didx(i)], ssems[b], add=True)
            pltpu.make_async_copy(rows_b[b], acc_sh.at[didx(i)], ssems[b]).wait()
            pltpu.async_copy(h_hbm.at[sidx(i + _NBUF)], rows_b[b], gsems[b])
        return 0

    lax.fori_loop(0, _STEADY_GROUPS, lambda g, c: step(g * _NBUF, c), 0)

    for b in range(_NBUF):
        i = _DRAIN_BASE - _NBUF + b
        pltpu.make_async_copy(h_hbm.at[sidx(i)], rows_b[b], gsems[b]).wait()
        pltpu.sync_copy(rows_b[b], acc_sh.at[didx(i)], add=True)

    pltpu.async_copy(h_hbm.at[sidx(_DRAIN_BASE)], rows_b[0], gsems[0])
    pltpu.make_async_copy(h_hbm.at[sidx(_DRAIN_BASE)], rows_b[0], gsems[0]).wait()
    pltpu.sync_copy(rows_b[0], acc_sh.at[didx(_DRAIN_BASE)], add=True)

    plsc.subcore_barrier()
    pltpu.sync_copy(acc_sh.at[pl.ds(sub * _ROWS_PER_TILE, _ROWS_PER_TILE)],
                    out_hbm.at[core, pl.ds(sub * _ROWS_PER_TILE, _ROWS_PER_TILE)])


def _sc_agg(h, src, dst, zeros_f):
    return pl.kernel(
        _sc_agg_body,
        out_type=jax.ShapeDtypeStruct((_NC, NP, D), jnp.float32),
        mesh=plsc.VectorSubcoreMesh(core_axis_name="c", subcore_axis_name="s", num_cores=_NC, num_subcores=_NS),
        scratch_types=(
            [pltpu.VMEM((_EDGES_PER_W,), jnp.int32) for _ in range(2)]
            + [pltpu.VMEM((_CHUNK, D), jnp.float32) for _ in range(_NBUF)]
            + [pltpu.VMEM_SHARED((NP, D), jnp.float32)]
            + [pltpu.SemaphoreType.DMA for _ in range(2 * _NBUF)]
        ),
    )(h, src, dst, zeros_f)


def _sc_score_gather_body(h_hbm, uidx_hbm, vidx_hbm, u_out, v_out,
                          uidx_v, vidx_v, urows_v, vrows_v, sem):
    wid = _wid()

    def step(i, _):
        base = pl.multiple_of(wid * (_SCORE_CHUNKS_PER_W * _CHUNK) + i * _CHUNK,
                              _CHUNK)
        pltpu.sync_copy(uidx_hbm.at[pl.ds(base, _CHUNK)], uidx_v)
        pltpu.sync_copy(vidx_hbm.at[pl.ds(base, _CHUNK)], vidx_v)
        pltpu.async_copy(h_hbm.at[uidx_v], urows_v, sem).wait()
        pltpu.async_copy(h_hbm.at[vidx_v], vrows_v, sem).wait()
        pltpu.sync_copy(urows_v, u_out.at[pl.ds(base, _CHUNK)])
        pltpu.sync_copy(vrows_v, v_out.at[pl.ds(base, _CHUNK)])
        return 0

    lax.fori_loop(0, _SCORE_CHUNKS_PER_W, step, 0)


def _sc_score_gather(h, uidx, vidx):
    return pl.kernel(
        _sc_score_gather_body,
        out_type=(jax.ShapeDtypeStruct((_SCORE_ROWS, D), jnp.float32),
                  jax.ShapeDtypeStruct((_SCORE_ROWS, D), jnp.float32)),
        mesh=plsc.VectorSubcoreMesh(core_axis_name="c", subcore_axis_name="s", num_cores=_NC, num_subcores=_NS),
        scratch_types=[
            pltpu.VMEM((_CHUNK,), jnp.int32),
            pltpu.VMEM((_CHUNK,), jnp.int32),
            pltpu.VMEM((_CHUNK, D), jnp.float32),
            pltpu.VMEM((_CHUNK, D), jnp.float32),
            pltpu.SemaphoreType.DMA,
        ],
    )(h, uidx, vidx)


_BN = 1024


def _tc_sage_body(relu, h_ref, p0_ref, p1_ref, c0_ref, c1_ref,
                  ws_ref, wn_ref, b_ref, out_ref):
    cnt = c0_ref[:, :1] + c1_ref[:, :1]
    inv = 1.0 / jnp.maximum(cnt, 1.0)
    agg = (p0_ref[...] + p1_ref[...]) * inv
    out = (jnp.dot(h_ref[...], ws_ref[...], preferred_element_type=jnp.float32)
           + jnp.dot(agg, wn_ref[...], preferred_element_type=jnp.float32)
           + b_ref[...])
    if relu:
        out = jnp.maximum(out, 0.0)
    out_ref[...] = out


def _tc_sage(h, parts, cnts, Ws, Wn, b, relu):
    grid = (NP // _BN,)
    return pl.pallas_call(
        functools.partial(_tc_sage_body, relu),
        grid=grid,
        in_specs=[
            pl.BlockSpec((_BN, D), lambda i: (i, 0)),
            pl.BlockSpec((_BN, D), lambda i: (i, 0)),
            pl.BlockSpec((_BN, D), lambda i: (i, 0)),
            pl.BlockSpec((_BN, 16), lambda i: (i, 0)),
            pl.BlockSpec((_BN, 16), lambda i: (i, 0)),
            pl.BlockSpec((D, D), lambda i: (0, 0)),
            pl.BlockSpec((D, D), lambda i: (0, 0)),
            pl.BlockSpec((1, D), lambda i: (0, 0)),
        ],
        out_specs=pl.BlockSpec((_BN, D), lambda i: (i, 0)),
        out_shape=jax.ShapeDtypeStruct((NP, D), jnp.float32),
    )(h, parts[0], parts[1], cnts[0], cnts[1], Ws, Wn, b)


_BS = 2048


def _tc_dot_body(u_ref, v_ref, out_ref):
    out_ref[...] = jnp.sum(u_ref[...] * v_ref[...], axis=-1, keepdims=True)


def _tc_dot(u_rows, v_rows):
    grid = (_SCORE_ROWS // _BS,)
    return pl.pallas_call(
        _tc_dot_body,
        grid=grid,
        in_specs=[
            pl.BlockSpec((_BS, D), lambda i: (i, 0)),
            pl.BlockSpec((_BS, D), lambda i: (i, 0)),
        ],
        out_specs=pl.BlockSpec((_BS, 1), lambda i: (i, 0)),
        out_shape=jax.ShapeDtypeStruct((_SCORE_ROWS, 1), jnp.float32),
    )(u_rows, v_rows)


def kernel(x, edge_index, pos_edge_index, neg_edge_index,
           W1n, W1s, b1, W2n, W2s, b2, W3n, W3s, b3):
    src = edge_index[0]
    dst = edge_index[1]
    zeros_f = jnp.zeros((NP, D), jnp.float32)
    ones_chunk = jnp.ones((_CHUNK, D), jnp.float32)

    cnt_parts = _sc_count(dst, ones_chunk, zeros_f)
    cnts = (cnt_parts[0][:, :16], cnt_parts[1][:, :16])

    h = jnp.pad(x, ((0, NP - N), (0, 0)))
    for (Wn, Ws, b, relu) in ((W1n, W1s, b1, True),
                              (W2n, W2s, b2, True),
                              (W3n, W3s, b3, False)):
        parts = _sc_agg(h, src, dst, zeros_f)
        h = _tc_sage(h, parts, cnts, Ws, Wn, b.reshape(1, D), relu)

    pad = jnp.zeros((_EPAD - pos_edge_index.shape[1],), jnp.int32)
    uidx = jnp.concatenate([pos_edge_index[0], pad, neg_edge_index[0], pad])
    vidx = jnp.concatenate([pos_edge_index[1], pad, neg_edge_index[1], pad])

    u_rows, v_rows = _sc_score_gather(h, uidx, vidx)
    scores = _tc_dot(u_rows, v_rows)

    pos_s = scores[:pos_edge_index.shape[1]]
    neg_s = scores[_EPAD:_EPAD + neg_edge_index.shape[1]]
    return (pos_s, neg_s)


# speedup vs baseline: 8.7808x; 1.1086x over previous
import functools

import jax
import jax.numpy as jnp
from jax import lax
from jax.experimental import pallas as pl
from jax.experimental.pallas import tpu as pltpu
from jax.experimental.pallas import tpu_sc as plsc

N = 10000
E = 320000
D = 128
NP = 10240

_NC = 2
_NS = 16
_NW = _NC * _NS

_CHUNK = 80
_EDGES_PER_W = E // _NW
_CHUNKS_PER_W = _EDGES_PER_W // _CHUNK

_EPAD = 10240
_SCORE_ROWS = 2 * _EPAD
_SCORE_CHUNKS_PER_W = _SCORE_ROWS // _NW // _CHUNK

_ROWS_PER_TILE = NP // _NS


def _wid():
    return lax.axis_index("s") * _NC + lax.axis_index("c")


_CGRP = 5
assert _CHUNKS_PER_W % _CGRP == 0


def _sc_count_body(dst_hbm, ones_hbm, zeros_hbm, out_hbm,
                   didx_all, ones_v, cnt_sh, ssem):
    wid = _wid()
    sub = lax.axis_index("s")
    core = lax.axis_index("c")
    ebase = pl.multiple_of(wid * _EDGES_PER_W, _EDGES_PER_W)

    pltpu.sync_copy(zeros_hbm.at[pl.ds(sub * _ROWS_PER_TILE, _ROWS_PER_TILE)],
                    cnt_sh.at[pl.ds(sub * _ROWS_PER_TILE, _ROWS_PER_TILE)])
    pltpu.sync_copy(ones_hbm, ones_v)
    pltpu.sync_copy(dst_hbm.at[pl.ds(ebase, _EDGES_PER_W)], didx_all)
    plsc.subcore_barrier()

    def didx(i):
        return didx_all.at[pl.ds(pl.multiple_of(i * _CHUNK, _CHUNK), _CHUNK)]

    def step(g, _):
        for b in range(_CGRP):
            pltpu.async_copy(ones_v, cnt_sh.at[didx(g + b)], ssem, add=True)
        for b in range(_CGRP):
            pltpu.make_async_copy(ones_v, cnt_sh.at[didx(g + b)], ssem).wait()
        return 0

    lax.fori_loop(0, _CHUNKS_PER_W // _CGRP, lambda g, c: step(g * _CGRP, c), 0)
    plsc.subcore_barrier()
    pltpu.sync_copy(cnt_sh.at[pl.ds(sub * _ROWS_PER_TILE, _ROWS_PER_TILE)],
                    out_hbm.at[core, pl.ds(sub * _ROWS_PER_TILE, _ROWS_PER_TILE)])


def _sc_count(dst, ones_chunk, zeros_f):
    return pl.kernel(
        _sc_count_body,
        out_type=jax.ShapeDtypeStruct((_NC, NP, D), jnp.float32),
        mesh=plsc.VectorSubcoreMesh(core_axis_name="c", subcore_axis_name="s", num_cores=_NC, num_subcores=_NS),
        scratch_types=[
            pltpu.VMEM((_EDGES_PER_W,), jnp.int32),
            pltpu.VMEM((_CHUNK, D), jnp.float32),
            pltpu.VMEM_SHARED((NP, D), jnp.float32),
            pltpu.SemaphoreType.DMA,
        ],
    )(dst, ones_chunk, zeros_f)


_NBUF = 2
_STEADY_GROUPS = (_CHUNKS_PER_W - 1) // _NBUF - 1
_DRAIN_BASE = _NBUF * (_STEADY_GROUPS + 1)


def _sc_agg_body(h_hbm, src_hbm, dst_hbm, zeros_hbm, out_hbm, *refs):
    sidx_all = refs[0]
    didx_all = refs[1]
    rows_b = refs[2:2 + _NBUF]
    acc_sh = refs[2 + _NBUF]
    gsems = refs[3 + _NBUF:3 + 2 * _NBUF]
    ssems = refs[3 + 2 * _NBUF:3 + 3 * _NBUF]

    wid = _wid()
    sub = lax.axis_index("s")
    core = lax.axis_index("c")
    ebase = pl.multiple_of(wid * _EDGES_PER_W, _EDGES_PER_W)

    pltpu.sync_copy(zeros_hbm.at[pl.ds(sub * _ROWS_PER_TILE, _ROWS_PER_TILE)],
                    acc_sh.at[pl.ds(sub * _ROWS_PER_TILE, _ROWS_PER_TILE)])
    pltpu.sync_copy(src_hbm.at[pl.ds(ebase, _EDGES_PER_W)], sidx_all)
    pltpu.sync_copy(dst_hbm.at[pl.ds(ebase, _EDGES_PER_W)], didx_all)
    plsc.subcore_barrier()

    def sidx(i):
        return sidx_all.at[pl.ds(pl.multiple_of(i * _CHUNK, _CHUNK), _CHUNK)]

    def didx(i):
        return didx_all.at[pl.ds(pl.multiple_of(i * _CHUNK, _CHUNK), _CHUNK)]

    for b in range(_NBUF):
        pltpu.async_copy(h_hbm.at[sidx(b)], rows_b[b], gsems[b])

    def step(g, _):
        for b in range(_NBUF):
            i = g + b
            pltpu.make_async_copy(h_hbm.at[sidx(i)], rows_b[b], gsems[b]).wait()
            pltpu.async_copy(rows_b[b], acc_sh.at[didx(i)], ssems[b], add=True)
            pltpu.make_async_copy(rows_b[b], acc_sh.at[didx(i)], ssems[b]).wait()
            pltpu.async_copy(h_hbm.at[sidx(i + _NBUF)], rows_b[b], gsems[b])
        return 0

    lax.fori_loop(0, _STEADY_GROUPS, lambda g, c: step(g * _NBUF, c), 0)

    for b in range(_NBUF):
        i = _DRAIN_BASE - _NBUF + b
        pltpu.make_async_copy(h_hbm.at[sidx(i)], rows_b[b], gsems[b]).wait()
        pltpu.sync_copy(rows_b[b], acc_sh.at[didx(i)], add=True)

    pltpu.async_copy(h_hbm.at[sidx(_DRAIN_BASE)], rows_b[0], gsems[0])
    pltpu.make_async_copy(h_hbm.at[sidx(_DRAIN_BASE)], rows_b[0], gsems[0]).wait()
    pltpu.sync_copy(rows_b[0], acc_sh.at[didx(_DRAIN_BASE)], add=True)

    plsc.subcore_barrier()
    pltpu.sync_copy(acc_sh.at[pl.ds(sub * _ROWS_PER_TILE, _ROWS_PER_TILE)],
                    out_hbm.at[core, pl.ds(sub * _ROWS_PER_TILE, _ROWS_PER_TILE)])


def _sc_agg(h, src, dst, zeros_f):
    return pl.kernel(
        _sc_agg_body,
        out_type=jax.ShapeDtypeStruct((_NC, NP, D), jnp.float32),
        mesh=plsc.VectorSubcoreMesh(core_axis_name="c", subcore_axis_name="s", num_cores=_NC, num_subcores=_NS),
        scratch_types=(
            [pltpu.VMEM((_EDGES_PER_W,), jnp.int32) for _ in range(2)]
            + [pltpu.VMEM((_CHUNK, D), jnp.float32) for _ in range(_NBUF)]
            + [pltpu.VMEM_SHARED((NP, D), jnp.float32)]
            + [pltpu.SemaphoreType.DMA for _ in range(2 * _NBUF)]
        ),
    )(h, src, dst, zeros_f)


_SCORE_PER_W = _SCORE_CHUNKS_PER_W * _CHUNK


def _sc_score_gather_body(h_hbm, uidx_hbm, vidx_hbm, u_out, v_out, *refs):
    uidx_all, vidx_all = refs[0], refs[1]
    urows = refs[2:4]
    vrows = refs[4:6]
    gsem_u = refs[6:8]
    gsem_v = refs[8:10]
    wsem_u = refs[10:12]
    wsem_v = refs[12:14]

    wid = _wid()
    sbase = pl.multiple_of(wid * _SCORE_PER_W, _SCORE_PER_W)
    pltpu.sync_copy(uidx_hbm.at[pl.ds(sbase, _SCORE_PER_W)], uidx_all)
    pltpu.sync_copy(vidx_hbm.at[pl.ds(sbase, _SCORE_PER_W)], vidx_all)

    def uidx(i):
        return uidx_all.at[pl.ds(pl.multiple_of(i * _CHUNK, _CHUNK), _CHUNK)]

    def vidx(i):
        return vidx_all.at[pl.ds(pl.multiple_of(i * _CHUNK, _CHUNK), _CHUNK)]

    def obase(i):
        return pl.multiple_of(sbase + i * _CHUNK, _CHUNK)

    for b in range(2):
        pltpu.async_copy(h_hbm.at[uidx(b)], urows[b], gsem_u[b])
        pltpu.async_copy(h_hbm.at[vidx(b)], vrows[b], gsem_v[b])

    for i in range(_SCORE_CHUNKS_PER_W):
        b = i % 2
        pltpu.make_async_copy(h_hbm.at[uidx(i)], urows[b], gsem_u[b]).wait()
        pltpu.async_copy(urows[b], u_out.at[pl.ds(obase(i), _CHUNK)], wsem_u[b])
        pltpu.make_async_copy(h_hbm.at[vidx(i)], vrows[b], gsem_v[b]).wait()
        pltpu.async_copy(vrows[b], v_out.at[pl.ds(obase(i), _CHUNK)], wsem_v[b])
        if i + 2 < _SCORE_CHUNKS_PER_W:
            pltpu.make_async_copy(urows[b], u_out.at[pl.ds(obase(i), _CHUNK)],
                                  wsem_u[b]).wait()
            pltpu.async_copy(h_hbm.at[uidx(i + 2)], urows[b], gsem_u[b])
            pltpu.make_async_copy(vrows[b], v_out.at[pl.ds(obase(i), _CHUNK)],
                                  wsem_v[b]).wait()
            pltpu.async_copy(h_hbm.at[vidx(i + 2)], vrows[b], gsem_v[b])

    for i in (_SCORE_CHUNKS_PER_W - 2, _SCORE_CHUNKS_PER_W - 1):
        b = i % 2
        pltpu.make_async_copy(urows[b], u_out.at[pl.ds(obase(i), _CHUNK)],
                              wsem_u[b]).wait()
        pltpu.make_async_copy(vrows[b], v_out.at[pl.ds(obase(i), _CHUNK)],
                              wsem_v[b]).wait()


def _sc_score_gather(h, uidx, vidx):
    return pl.kernel(
        _sc_score_gather_body,
        out_type=(jax.ShapeDtypeStruct((_SCORE_ROWS, D), jnp.float32),
                  jax.ShapeDtypeStruct((_SCORE_ROWS, D), jnp.float32)),
        mesh=plsc.VectorSubcoreMesh(core_axis_name="c", subcore_axis_name="s", num_cores=_NC, num_subcores=_NS),
        scratch_types=(
            [pltpu.VMEM((_SCORE_PER_W,), jnp.int32) for _ in range(2)]
            + [pltpu.VMEM((_CHUNK, D), jnp.float32) for _ in range(4)]
            + [pltpu.SemaphoreType.DMA for _ in range(8)]
        ),
    )(h, uidx, vidx)


_BN = 1024


def _tc_sage_body(relu, h_ref, p0_ref, p1_ref, c0_ref, c1_ref,
                  ws_ref, wn_ref, b_ref, out_ref):
    cnt = c0_ref[:, :1] + c1_ref[:, :1]
    inv = 1.0 / jnp.maximum(cnt, 1.0)
    agg = (p0_ref[...] + p1_ref[...]) * inv
    out = (jnp.dot(h_ref[...], ws_ref[...], preferred_element_type=jnp.float32)
           + jnp.dot(agg, wn_ref[...], preferred_element_type=jnp.float32)
           + b_ref[...])
    if relu:
        out = jnp.maximum(out, 0.0)
    out_ref[...] = out


def _tc_sage(h, parts, cnts, Ws, Wn, b, relu):
    grid = (NP // _BN,)
    return pl.pallas_call(
        functools.partial(_tc_sage_body, relu),
        grid=grid,
        in_specs=[
            pl.BlockSpec((_BN, D), lambda i: (i, 0)),
            pl.BlockSpec((_BN, D), lambda i: (i, 0)),
            pl.BlockSpec((_BN, D), lambda i: (i, 0)),
            pl.BlockSpec((_BN, 16), lambda i: (i, 0)),
            pl.BlockSpec((_BN, 16), lambda i: (i, 0)),
            pl.BlockSpec((D, D), lambda i: (0, 0)),
            pl.BlockSpec((D, D), lambda i: (0, 0)),
            pl.BlockSpec((1, D), lambda i: (0, 0)),
        ],
        out_specs=pl.BlockSpec((_BN, D), lambda i: (i, 0)),
        out_shape=jax.ShapeDtypeStruct((NP, D), jnp.float32),
    )(h, parts[0], parts[1], cnts[0], cnts[1], Ws, Wn, b)


_BS = 2048


def _tc_dot_body(u_ref, v_ref, out_ref):
    out_ref[...] = jnp.sum(u_ref[...] * v_ref[...], axis=-1, keepdims=True)


def _tc_dot(u_rows, v_rows):
    grid = (_SCORE_ROWS // _BS,)
    return pl.pallas_call(
        _tc_dot_body,
        grid=grid,
        in_specs=[
            pl.BlockSpec((_BS, D), lambda i: (i, 0)),
            pl.BlockSpec((_BS, D), lambda i: (i, 0)),
        ],
        out_specs=pl.BlockSpec((_BS, 1), lambda i: (i, 0)),
        out_shape=jax.ShapeDtypeStruct((_SCORE_ROWS, 1), jnp.float32),
    )(u_rows, v_rows)


def kernel(x, edge_index, pos_edge_index, neg_edge_index,
           W1n, W1s, b1, W2n, W2s, b2, W3n, W3s, b3):
    src = edge_index[0]
    dst = edge_index[1]
    zeros_f = jnp.zeros((NP, D), jnp.float32)
    ones_chunk = jnp.ones((_CHUNK, D), jnp.float32)

    cnt_parts = _sc_count(dst, ones_chunk, zeros_f)
    cnts = (cnt_parts[0][:, :16], cnt_parts[1][:, :16])

    h = jnp.pad(x, ((0, NP - N), (0, 0)))
    for (Wn, Ws, b, relu) in ((W1n, W1s, b1, True),
                              (W2n, W2s, b2, True),
                              (W3n, W3s, b3, False)):
        parts = _sc_agg(h, src, dst, zeros_f)
        h = _tc_sage(h, parts, cnts, Ws, Wn, b.reshape(1, D), relu)

    pad = jnp.zeros((_EPAD - pos_edge_index.shape[1],), jnp.int32)
    uidx = jnp.concatenate([pos_edge_index[0], pad, neg_edge_index[0], pad])
    vidx = jnp.concatenate([pos_edge_index[1], pad, neg_edge_index[1], pad])

    u_rows, v_rows = _sc_score_gather(h, uidx, vidx)
    scores = _tc_dot(u_rows, v_rows)

    pos_s = scores[:pos_edge_index.shape[1]]
    neg_s = scores[_EPAD:_EPAD + neg_edge_index.shape[1]]
    return (pos_s, neg_s)
